# baseline (speedup 1.0000x reference)
import functools

import jax
import jax.numpy as jnp
from jax import lax
from jax.experimental import pallas as pl
from jax.experimental.pallas import tpu as pltpu
from jax.experimental.pallas import tpu_sc as plsc

NUM_ROWS = 16384
NUM_COLS = 26
DIM = 64
B = NUM_ROWS * NUM_COLS

_info = plsc.get_sparse_core_info()
NC = _info.num_cores
NS = _info.num_subcores
NW = NC * NS
ROWS_PER_W = NUM_ROWS // NW
B_PER_W = ROWS_PER_W * NUM_COLS
CROWS = 16
CHUNK = CROWS * NUM_COLS
N_CHUNKS = ROWS_PER_W // CROWS
NBUF = 3
LANES = 16

assert N_CHUNKS * CROWS == ROWS_PER_W


def _body(x_hbm, w_hbm, out_hbm, idx2d_v, idx_v, rows_v, *sems):
    gsem = sems[:NBUF]
    osem = sems[NBUF:]
    wid = lax.axis_index("s") * NC + lax.axis_index("c")
    row0 = wid * ROWS_PER_W

    pltpu.sync_copy(x_hbm.at[pl.ds(row0, ROWS_PER_W)], idx2d_v)

    lane = lax.broadcasted_iota(jnp.int32, (LANES,), 0)

    def flatten_step(t, _):
        p = t * LANES + lane
        r = p // NUM_COLS
        c = p % NUM_COLS
        v = plsc.load_gather(idx2d_v, [r, c])
        idx_v[t // NUM_COLS, pl.ds((t % NUM_COLS) * LANES, LANES)] = v
        return ()

    lax.fori_loop(0, B_PER_W // LANES, flatten_step, ())

    def fire_gather(g, b):
        pltpu.async_copy(w_hbm.at[idx_v.at[g]], rows_v.at[b], gsem[b])

    def wait_gather(b):
        pltpu.make_async_copy(
            w_hbm.at[idx_v.at[0]], rows_v.at[b], gsem[b]
        ).wait()

    def fire_out(g, b):
        for i in range(CROWS):
            pltpu.async_copy(
                rows_v.at[b, pl.ds(i * NUM_COLS, NUM_COLS)],
                out_hbm.at[row0 + g * CROWS + i],
                osem[b],
            )

    def wait_out(b):
        for i in range(CROWS):
            pltpu.make_async_copy(
                rows_v.at[b, pl.ds(i * NUM_COLS, NUM_COLS)],
                out_hbm.at[row0],
                osem[b],
            ).wait()

    for g in range(NBUF - 1):
        fire_gather(g, g % NBUF)

    def step(g, b, first):
        wait_gather(b)
        fire_out(g, b)
        bf = (b + NBUF - 1) % NBUF
        if not first:
            wait_out(bf)
        fire_gather(g + NBUF - 1, bf)

    for g in range(NBUF):
        step(g, g % NBUF, first=(g == 0))

    n_steady_groups = (N_CHUNKS - (NBUF - 1)) // NBUF - 1

    def steady(i, _):
        g0 = (i + 1) * NBUF
        for b in range(NBUF):
            step(g0 + b, b, first=False)
        return ()

    lax.fori_loop(0, n_steady_groups, steady, ())

    tail_start = (n_steady_groups + 1) * NBUF
    for g in range(tail_start, N_CHUNKS):
        b = g % NBUF
        wait_gather(b)
        fire_out(g, b)
    for b in range(NBUF):
        wait_out(b)


def kernel(x, weight):
    mesh = plsc.VectorSubcoreMesh(core_axis_name="c", subcore_axis_name="s")
    run = functools.partial(
        pl.kernel,
        mesh=mesh,
        out_type=jax.ShapeDtypeStruct((NUM_ROWS, NUM_COLS, DIM), jnp.float32),
        scratch_types=[
            pltpu.VMEM((ROWS_PER_W, NUM_COLS), jnp.int32),
            pltpu.VMEM((N_CHUNKS, CHUNK), jnp.int32),
            pltpu.VMEM((NBUF, CHUNK, DIM), jnp.float32),
        ]
        + [pltpu.SemaphoreType.DMA] * (2 * NBUF),
        compiler_params=pltpu.CompilerParams(
            use_tc_tiling_on_sc=False, needs_layout_passes=False
        ),
    )(_body)
    return run(x, weight)


# speedup vs baseline: 1.0705x; 1.0705x over previous
import functools

import jax
import jax.numpy as jnp
from jax import lax
from jax.experimental import pallas as pl
from jax.experimental.pallas import tpu as pltpu
from jax.experimental.pallas import tpu_sc as plsc

NUM_ROWS = 16384
NUM_COLS = 26
NUM_EMB = 1000000
DIM = 64
B = NUM_ROWS * NUM_COLS

_info = plsc.get_sparse_core_info()
NC = _info.num_cores
NS = _info.num_subcores
NW = NC * NS
ROWS_PER_W = NUM_ROWS // NW
B_PER_W = ROWS_PER_W * NUM_COLS
CROWS = 16
CHUNK = CROWS * NUM_COLS
N_CHUNKS = ROWS_PER_W // CROWS
NBUF = 3
LANES = 16
PDIM = 128

assert N_CHUNKS * CROWS == ROWS_PER_W


def _body(x_hbm, w_hbm, out_hbm, idx2d_v, idx_v, rows_v, *sems):
    gsem = sems[:NBUF]
    osem = sems[NBUF:]
    wid = lax.axis_index("s") * NC + lax.axis_index("c")
    row0 = wid * ROWS_PER_W

    pltpu.sync_copy(x_hbm.at[pl.ds(row0, ROWS_PER_W)], idx2d_v)

    lane = lax.broadcasted_iota(jnp.int32, (LANES,), 0)

    def flatten_step(t, _):
        p = t * LANES + lane
        r = p // NUM_COLS
        c = p % NUM_COLS
        v = plsc.load_gather(idx2d_v, [r, c])
        idx_v[t // NUM_COLS, pl.ds((t % NUM_COLS) * LANES, LANES)] = v + v
        return ()

    lax.fori_loop(0, B_PER_W // LANES, flatten_step, ())

    def fire_gather(g, b):
        pltpu.async_copy(w_hbm.at[idx_v.at[g]], rows_v.at[b], gsem[b])

    def wait_gather(b):
        pltpu.make_async_copy(
            w_hbm.at[idx_v.at[0]], rows_v.at[b], gsem[b]
        ).wait()

    def fire_out(g, b):
        for i in range(CROWS):
            pltpu.async_copy(
                rows_v.at[b, pl.ds(i * NUM_COLS, NUM_COLS)],
                out_hbm.at[row0 + g * CROWS + i],
                osem[b],
            )

    def wait_out(b):
        for i in range(CROWS):
            pltpu.make_async_copy(
                rows_v.at[b, pl.ds(i * NUM_COLS, NUM_COLS)],
                out_hbm.at[row0],
                osem[b],
            ).wait()

    for g in range(NBUF - 1):
        fire_gather(g, g % NBUF)

    def step(g, b, first):
        wait_gather(b)
        fire_out(g, b)
        bf = (b + NBUF - 1) % NBUF
        if not first:
            wait_out(bf)
        fire_gather(g + NBUF - 1, bf)

    for g in range(NBUF):
        step(g, g % NBUF, first=(g == 0))

    n_steady_groups = (N_CHUNKS - (NBUF - 1)) // NBUF - 1

    def steady(i, _):
        g0 = (i + 1) * NBUF
        for b in range(NBUF):
            step(g0 + b, b, first=False)
        return ()

    lax.fori_loop(0, n_steady_groups, steady, ())

    tail_start = (n_steady_groups + 1) * NBUF
    for g in range(tail_start, N_CHUNKS):
        b = g % NBUF
        wait_gather(b)
        fire_out(g, b)
    for b in range(NBUF):
        wait_out(b)


def kernel(x, weight):
    wp = jnp.pad(weight.reshape(NUM_EMB // 8, 8, DIM), ((0, 0), (0, 0), (0, PDIM - DIM)))
    wp = wp.reshape(2 * NUM_EMB, DIM)
    mesh = plsc.VectorSubcoreMesh(core_axis_name="c", subcore_axis_name="s")
    run = functools.partial(
        pl.kernel,
        mesh=mesh,
        out_type=jax.ShapeDtypeStruct((NUM_ROWS, NUM_COLS, DIM), jnp.float32),
        scratch_types=[
            pltpu.VMEM((ROWS_PER_W, NUM_COLS), jnp.int32),
            pltpu.VMEM((N_CHUNKS, CHUNK), jnp.int32),
            pltpu.VMEM((NBUF, CHUNK, DIM), jnp.float32),
        ]
        + [pltpu.SemaphoreType.DMA] * (2 * NBUF),
        compiler_params=pltpu.CompilerParams(
            use_tc_tiling_on_sc=False, needs_layout_passes=False
        ),
    )(_body)
    return run(x, wp)
